# baseline (speedup 1.0000x reference)
import functools
import math

import jax
import jax.numpy as jnp
from jax import lax
from jax.experimental import pallas as pl
from jax.experimental.pallas import tpu as pltpu
from jax.experimental.pallas import tpu_sc as plsc

MAX_DIST = 10.0
GRID_RESOLUTION = 1.0
BOX = int(math.ceil(2 * MAX_DIST / GRID_RESOLUTION + 1))
NCELL = BOX * BOX * BOX
NROW = 9344
DUMMY = NCELL
CHUNK = 128
NPG = 8
NSLAB = 4
FS = 8
MAGIC = 12582912.0
SHIFT = MAGIC + MAX_DIST


def _sc_scatter(coords_flat, feats_flat, n, f):
  pg_size = n // NPG
  nchunk = (pg_size + CHUNK - 1) // CHUNK
  mesh = plsc.VectorSubcoreMesh(core_axis_name="c", subcore_axis_name="s")

  @functools.partial(
      pl.kernel,
      out_type=jax.ShapeDtypeStruct((NSLAB * NPG, NROW * FS), jnp.float32),
      mesh=mesh,
      scratch_types=[
          pltpu.VMEM((CHUNK * 3,), jnp.float32),
          pltpu.VMEM((CHUNK * 32,), jnp.float32),
          pltpu.VMEM((NROW * FS,), jnp.float32),
      ],
      compiler_params=pltpu.CompilerParams(needs_layout_passes=False),
  )
  def k(coords_h, feats_h, out_h, cbuf, fbuf, slab):
    cid = lax.axis_index("c")
    sid = lax.axis_index("s")
    wid = cid * 16 + sid
    s = wid // NPG
    pg = wid % NPG
    col0 = s * FS

    zeros16 = jnp.zeros((16,), jnp.float32)

    def zrow(i, carry):
      slab[pl.ds(i * 16, 16)] = zeros16
      return carry

    lax.fori_loop(0, NROW * FS // 16, zrow, 0)

    lanes = lax.iota(jnp.int32, 16)
    pg_start = pg * pg_size
    pg_end = pg_start + pg_size

    def body(c, carry):
      base_l = pg_start + c * CHUNK
      base = jnp.minimum(base_l, n - CHUNK)
      pltpu.sync_copy(coords_h.at[pl.ds(base * 3, CHUNK * 3)], cbuf)
      pltpu.sync_copy(feats_h.at[pl.ds(base * f, CHUNK * f)], fbuf)
      for j in range(CHUNK // 16):
        row = j * 16 + lanes
        grow = base + row
        valid = (grow >= base_l) & (grow < pg_end)
        row3 = row * 3
        x = plsc.load_gather(cbuf, [row3])
        y = plsc.load_gather(cbuf, [row3 + 1])
        z = plsc.load_gather(cbuf, [row3 + 2])
        rx = (x + SHIFT) - MAGIC
        ry = (y + SHIFT) - MAGIC
        rz = (z + SHIFT) - MAGIC
        cx = jnp.minimum(jnp.maximum(rx, 0.0), float(BOX - 1))
        cy = jnp.minimum(jnp.maximum(ry, 0.0), float(BOX - 1))
        cz = jnp.minimum(jnp.maximum(rz, 0.0), float(BOX - 1))
        ok = (rx == cx) & (ry == cy) & (rz == cz) & valid
        cell = (cx * float(BOX * BOX) + cy * float(BOX) + cz).astype(jnp.int32)
        sidx = jnp.where(ok, cell, DUMMY) * FS
        rowf = row * f + col0
        for kk in range(FS):
          val = plsc.load_gather(fbuf, [rowf + kk])
          plsc.addupdate_scatter(slab, [sidx + kk], val)
      return carry

    lax.fori_loop(0, nchunk, body, 0)

    pltpu.sync_copy(slab, out_h.at[wid])

  return k(coords_flat, feats_flat)


def _combine_body(p_ref, o_ref):
  for s in range(NSLAB):
    a = p_ref[s * NPG]
    for pg in range(1, NPG):
      a = a + p_ref[s * NPG + pg]
    o_ref[s] = a


def kernel(coords, features):
  b, n, _ = coords.shape
  f = features.shape[2]
  coords_flat = coords.reshape(n * 3)
  feats_flat = features.reshape(n * f)
  partials = _sc_scatter(coords_flat, feats_flat, n, f)
  lanes = NROW * FS // 128
  summed = pl.pallas_call(
      _combine_body,
      out_shape=jax.ShapeDtypeStruct((NSLAB, lanes, 128), jnp.float32),
  )(partials.reshape(NSLAB * NPG, lanes, 128))
  grid2d = summed.reshape(NSLAB, NROW, FS).transpose(1, 0, 2).reshape(NROW, f)
  return grid2d[:NCELL].reshape(b, BOX, BOX, BOX, f)


# speedup vs baseline: 1.2320x; 1.2320x over previous
import functools
import math

import jax
import jax.numpy as jnp
from jax import lax
from jax.experimental import pallas as pl
from jax.experimental.pallas import tpu as pltpu
from jax.experimental.pallas import tpu_sc as plsc

MAX_DIST = 10.0
GRID_RESOLUTION = 1.0
BOX = int(math.ceil(2 * MAX_DIST / GRID_RESOLUTION + 1))
NCELL = BOX * BOX * BOX
NROW = 9344
DUMMY = NCELL
CHUNK = 256
NPG = 8
NSLAB = 4
FS = 8
MAGIC = 12582912.0
SHIFT = MAGIC + MAX_DIST


def _sc_scatter(coords_flat, feats_flat, n, f):
  pg_size = n // NPG
  nchunk = (pg_size + CHUNK - 1) // CHUNK
  mesh = plsc.VectorSubcoreMesh(core_axis_name="c", subcore_axis_name="s")

  @functools.partial(
      pl.kernel,
      out_type=jax.ShapeDtypeStruct((NSLAB * NPG, NROW * FS), jnp.float32),
      mesh=mesh,
      scratch_types=[
          pltpu.VMEM((CHUNK * 3,), jnp.float32),
          pltpu.VMEM((CHUNK * 3,), jnp.float32),
          pltpu.VMEM((CHUNK * 32,), jnp.float32),
          pltpu.VMEM((CHUNK * 32,), jnp.float32),
          pltpu.VMEM((NROW * FS,), jnp.float32),
          pltpu.SemaphoreType.DMA,
          pltpu.SemaphoreType.DMA,
      ],
      compiler_params=pltpu.CompilerParams(needs_layout_passes=False),
  )
  def k(coords_h, feats_h, out_h, cbuf0, cbuf1, fbuf0, fbuf1, slab, sem0,
        sem1):
    cid = lax.axis_index("c")
    sid = lax.axis_index("s")
    wid = cid * 16 + sid
    s = wid // NPG
    pg = wid % NPG
    col0 = s * FS

    lanes = lax.iota(jnp.int32, 16)
    pg_start = pg * pg_size
    pg_end = pg_start + pg_size
    bufs = ((cbuf0, fbuf0, sem0), (cbuf1, fbuf1, sem1))

    def start(c, b):
      cb, fb, sem = bufs[b]
      base = jnp.minimum(pg_start + c * CHUNK, n - CHUNK)
      pltpu.async_copy(coords_h.at[pl.ds(base * 3, CHUNK * 3)], cb, sem)
      pltpu.async_copy(feats_h.at[pl.ds(base * f, CHUNK * f)], fb, sem)

    def wait(b):
      cb, fb, sem = bufs[b]
      pltpu.make_async_copy(coords_h.at[pl.ds(0, CHUNK * 3)], cb, sem).wait()
      pltpu.make_async_copy(feats_h.at[pl.ds(0, CHUNK * f)], fb, sem).wait()

    def compute(c, b):
      cb, fb, _ = bufs[b]
      base_l = pg_start + c * CHUNK
      base = jnp.minimum(base_l, n - CHUNK)
      for j in range(CHUNK // 16):
        row = j * 16 + lanes
        grow = base + row
        valid = (grow >= base_l) & (grow < pg_end)
        row3 = row * 3
        x = plsc.load_gather(cb, [row3])
        y = plsc.load_gather(cb, [row3 + 1])
        z = plsc.load_gather(cb, [row3 + 2])
        rx = (x + SHIFT) - MAGIC
        ry = (y + SHIFT) - MAGIC
        rz = (z + SHIFT) - MAGIC
        cx = jnp.minimum(jnp.maximum(rx, 0.0), float(BOX - 1))
        cy = jnp.minimum(jnp.maximum(ry, 0.0), float(BOX - 1))
        cz = jnp.minimum(jnp.maximum(rz, 0.0), float(BOX - 1))
        ok = (rx == cx) & (ry == cy) & (rz == cz) & valid
        cell = (cx * float(BOX * BOX) + cy * float(BOX) + cz).astype(jnp.int32)
        sidx = jnp.where(ok, cell, DUMMY) * FS
        rowf = row * f + col0
        for kk in range(FS):
          val = plsc.load_gather(fb, [rowf + kk])
          plsc.addupdate_scatter(slab, [sidx + kk], val)

    start(jnp.int32(0), 0)

    zeros16 = jnp.zeros((16,), jnp.float32)

    def zrow(i, carry):
      slab[pl.ds(i * 16, 16)] = zeros16
      return carry

    lax.fori_loop(0, NROW * FS // 16, zrow, 0)

    npair = (nchunk + 1) // 2

    def body(g, carry):
      c = g * 2

      @pl.when(c + 1 < nchunk)
      def _():
        start(c + 1, 1)

      wait(0)
      compute(c, 0)

      @pl.when(c + 2 < nchunk)
      def _():
        start(c + 2, 0)

      @pl.when(c + 1 < nchunk)
      def _():
        wait(1)
        compute(c + 1, 1)

      return carry

    lax.fori_loop(0, npair, body, 0)

    pltpu.sync_copy(slab, out_h.at[wid])

  return k(coords_flat, feats_flat)


def _combine_body(p_ref, o_ref):
  for s in range(NSLAB):
    a = p_ref[s * NPG]
    for pg in range(1, NPG):
      a = a + p_ref[s * NPG + pg]
    o_ref[s] = a


def kernel(coords, features):
  b, n, _ = coords.shape
  f = features.shape[2]
  coords_flat = coords.reshape(n * 3)
  feats_flat = features.reshape(n * f)
  partials = _sc_scatter(coords_flat, feats_flat, n, f)
  lanes = NROW * FS // 128
  summed = pl.pallas_call(
      _combine_body,
      out_shape=jax.ShapeDtypeStruct((NSLAB, lanes, 128), jnp.float32),
  )(partials.reshape(NSLAB * NPG, lanes, 128))
  grid2d = summed.reshape(NSLAB, NROW, FS).transpose(1, 0, 2).reshape(NROW, f)
  return grid2d[:NCELL].reshape(b, BOX, BOX, BOX, f)


# speedup vs baseline: 1.2828x; 1.0412x over previous
import functools
import math

import jax
import jax.numpy as jnp
from jax import lax
from jax.experimental import pallas as pl
from jax.experimental.pallas import tpu as pltpu
from jax.experimental.pallas import tpu_sc as plsc

MAX_DIST = 10.0
GRID_RESOLUTION = 1.0
BOX = int(math.ceil(2 * MAX_DIST / GRID_RESOLUTION + 1))
NCELL = BOX * BOX * BOX
NROW = 9344
DUMMY = NCELL
CHUNK = 256
NPG = 8
NSLAB = 4
FS = 8
MAGIC = 12582912.0
SHIFT = MAGIC + MAX_DIST


def _sc_scatter(coords_flat, feats_flat, n, f):
  pg_size = n // NPG
  nchunk = (pg_size + CHUNK - 1) // CHUNK
  mesh = plsc.VectorSubcoreMesh(core_axis_name="c", subcore_axis_name="s")

  @functools.partial(
      pl.kernel,
      out_type=jax.ShapeDtypeStruct((NPG, 32, NROW), jnp.float32),
      mesh=mesh,
      scratch_types=[
          pltpu.VMEM((CHUNK * 3,), jnp.float32),
          pltpu.VMEM((CHUNK * 3,), jnp.float32),
          pltpu.VMEM((CHUNK * 32,), jnp.float32),
          pltpu.VMEM((CHUNK * 32,), jnp.float32),
          pltpu.VMEM((FS, NROW), jnp.float32),
          pltpu.SemaphoreType.DMA,
          pltpu.SemaphoreType.DMA,
      ],
      compiler_params=pltpu.CompilerParams(needs_layout_passes=False),
  )
  def k(coords_h, feats_h, out_h, cbuf0, cbuf1, fbuf0, fbuf1, slab, sem0,
        sem1):
    cid = lax.axis_index("c")
    sid = lax.axis_index("s")
    wid = cid * 16 + sid
    s = wid // NPG
    pg = wid % NPG
    col0 = s * FS

    lanes = lax.iota(jnp.int32, 16)
    kkvecs = [jnp.full((16,), kk, jnp.int32) for kk in range(FS)]
    pg_start = pg * pg_size
    pg_end = pg_start + pg_size
    bufs = ((cbuf0, fbuf0, sem0), (cbuf1, fbuf1, sem1))

    def start(c, b):
      cb, fb, sem = bufs[b]
      base = jnp.minimum(pg_start + c * CHUNK, n - CHUNK)
      pltpu.async_copy(coords_h.at[pl.ds(base * 3, CHUNK * 3)], cb, sem)
      pltpu.async_copy(feats_h.at[pl.ds(base * f, CHUNK * f)], fb, sem)

    def wait(b):
      cb, fb, sem = bufs[b]
      pltpu.make_async_copy(coords_h.at[pl.ds(0, CHUNK * 3)], cb, sem).wait()
      pltpu.make_async_copy(feats_h.at[pl.ds(0, CHUNK * f)], fb, sem).wait()

    def compute(c, b):
      cb, fb, _ = bufs[b]
      base_l = pg_start + c * CHUNK
      base = jnp.minimum(base_l, n - CHUNK)
      for j in range(CHUNK // 16):
        row = j * 16 + lanes
        grow = base + row
        valid = (grow >= base_l) & (grow < pg_end)
        row3 = row * 3
        x = plsc.load_gather(cb, [row3])
        y = plsc.load_gather(cb, [row3 + 1])
        z = plsc.load_gather(cb, [row3 + 2])
        rx = (x + SHIFT) - MAGIC
        ry = (y + SHIFT) - MAGIC
        rz = (z + SHIFT) - MAGIC
        cx = jnp.minimum(jnp.maximum(rx, 0.0), float(BOX - 1))
        cy = jnp.minimum(jnp.maximum(ry, 0.0), float(BOX - 1))
        cz = jnp.minimum(jnp.maximum(rz, 0.0), float(BOX - 1))
        ok = (rx == cx) & (ry == cy) & (rz == cz) & valid
        cell = (cx * float(BOX * BOX) + cy * float(BOX) + cz).astype(jnp.int32)
        sidx = jnp.where(ok, cell, DUMMY)
        rowf = row * f + col0
        for kk in range(FS):
          val = plsc.load_gather(fb, [rowf + kk])
          plsc.addupdate_scatter(slab, [kkvecs[kk], sidx], val)

    start(jnp.int32(0), 0)

    zeros16 = jnp.zeros((16,), jnp.float32)

    def zrow(i, carry):
      for kk in range(FS):
        slab[kk, pl.ds(i * 16, 16)] = zeros16
      return carry

    lax.fori_loop(0, NROW // 16, zrow, 0)

    npair = (nchunk + 1) // 2

    def body(g, carry):
      c = g * 2

      @pl.when(c + 1 < nchunk)
      def _():
        start(c + 1, 1)

      wait(0)
      compute(c, 0)

      @pl.when(c + 2 < nchunk)
      def _():
        start(c + 2, 0)

      @pl.when(c + 1 < nchunk)
      def _():
        wait(1)
        compute(c + 1, 1)

      return carry

    lax.fori_loop(0, npair, body, 0)

    pltpu.sync_copy(slab, out_h.at[pg, pl.ds(col0, FS), :])

  return k(coords_flat, feats_flat)


def _combine_body(p_ref, o_ref):
  a = p_ref[0]
  for pg in range(1, NPG):
    a = a + p_ref[pg]
  o_ref[...] = a.T


def kernel(coords, features):
  b, n, _ = coords.shape
  f = features.shape[2]
  coords_flat = coords.reshape(n * 3)
  feats_flat = features.reshape(n * f)
  partials = _sc_scatter(coords_flat, feats_flat, n, f)
  summed = pl.pallas_call(
      _combine_body,
      out_shape=jax.ShapeDtypeStruct((NROW, f), jnp.float32),
  )(partials)
  return summed[:NCELL].reshape(b, BOX, BOX, BOX, f)


# speedup vs baseline: 2.0008x; 1.5597x over previous
import functools
import math

import jax
import jax.numpy as jnp
from jax import lax
from jax.experimental import pallas as pl
from jax.experimental.pallas import tpu as pltpu
from jax.experimental.pallas import tpu_sc as plsc

MAX_DIST = 10.0
GRID_RESOLUTION = 1.0
BOX = int(math.ceil(2 * MAX_DIST / GRID_RESOLUTION + 1))
NCELL = BOX * BOX * BOX
NROW = 9344
DUMMY = NCELL
CHUNK = 256
NPG = 8
NSLAB = 4
FS = 8
MAGIC = 12582912.0


def _sc_scatter(xs, ys, zs, ftf, n, f):
  nfull = n // CHUNK
  tail = n - nfull * CHUNK
  assert tail % 16 == 0 and CHUNK % 16 == 0
  tail_pg = nfull % NPG
  max_m = (nfull + NPG - 1) // NPG
  npair = (max_m + 1) // 2
  mesh = plsc.VectorSubcoreMesh(core_axis_name="c", subcore_axis_name="s")

  @functools.partial(
      pl.kernel,
      out_type=jax.ShapeDtypeStruct((NPG, f, NROW), jnp.float32),
      mesh=mesh,
      scratch_types=[
          pltpu.VMEM((CHUNK,), jnp.float32),
          pltpu.VMEM((CHUNK,), jnp.float32),
          pltpu.VMEM((CHUNK,), jnp.float32),
          pltpu.VMEM((FS * CHUNK,), jnp.float32),
          pltpu.VMEM((CHUNK,), jnp.float32),
          pltpu.VMEM((CHUNK,), jnp.float32),
          pltpu.VMEM((CHUNK,), jnp.float32),
          pltpu.VMEM((FS * CHUNK,), jnp.float32),
          pltpu.VMEM((FS, NROW), jnp.float32),
          pltpu.SemaphoreType.DMA,
          pltpu.SemaphoreType.DMA,
      ],
      compiler_params=pltpu.CompilerParams(needs_layout_passes=False),
  )
  def k(xs_h, ys_h, zs_h, ftf_h, out_h, xb0, yb0, zb0, fb0, xb1, yb1, zb1,
        fb1, slab, sem0, sem1):
    cid = lax.axis_index("c")
    sid = lax.axis_index("s")
    wid = cid * 16 + sid
    s = wid // NPG
    pg = wid % NPG
    col0 = s * FS

    kkvecs = [jnp.full((16,), kk, jnp.int32) for kk in range(FS)]
    bufs = ((xb0, yb0, zb0, fb0, sem0), (xb1, yb1, zb1, fb1, sem1))
    m = (nfull - pg + NPG - 1) // NPG

    def start(kc, b):
      xb, yb, zb, fb, sem = bufs[b]
      base = (pg + kc * NPG) * CHUNK
      pltpu.async_copy(xs_h.at[pl.ds(base, CHUNK)], xb, sem)
      pltpu.async_copy(ys_h.at[pl.ds(base, CHUNK)], yb, sem)
      pltpu.async_copy(zs_h.at[pl.ds(base, CHUNK)], zb, sem)
      for kk in range(FS):
        pltpu.async_copy(ftf_h.at[pl.ds((col0 + kk) * n + base, CHUNK)],
                         fb.at[pl.ds(kk * CHUNK, CHUNK)], sem)

    def wait(b):
      xb, yb, zb, fb, sem = bufs[b]
      pltpu.make_async_copy(xs_h.at[pl.ds(0, CHUNK)], xb, sem).wait()
      pltpu.make_async_copy(ys_h.at[pl.ds(0, CHUNK)], yb, sem).wait()
      pltpu.make_async_copy(zs_h.at[pl.ds(0, CHUNK)], zb, sem).wait()
      for kk in range(FS):
        pltpu.make_async_copy(ftf_h.at[pl.ds(0, CHUNK)],
                              fb.at[pl.ds(kk * CHUNK, CHUNK)], sem).wait()

    def step(b, j):
      xb, yb, zb, fb, _ = bufs[b]
      x = xb[pl.ds(j * 16, 16)]
      y = yb[pl.ds(j * 16, 16)]
      z = zb[pl.ds(j * 16, 16)]
      rx = ((x + MAX_DIST) + MAGIC) - MAGIC
      ry = ((y + MAX_DIST) + MAGIC) - MAGIC
      rz = ((z + MAX_DIST) + MAGIC) - MAGIC
      cx = jnp.minimum(jnp.maximum(rx, 0.0), float(BOX - 1))
      cy = jnp.minimum(jnp.maximum(ry, 0.0), float(BOX - 1))
      cz = jnp.minimum(jnp.maximum(rz, 0.0), float(BOX - 1))
      ok = (rx == cx) & (ry == cy) & (rz == cz)
      cell = (cx * float(BOX * BOX) + cy * float(BOX) + cz).astype(jnp.int32)
      sidx = jnp.where(ok, cell, DUMMY)
      for kk in range(FS):
        val = fb[pl.ds(kk * CHUNK + j * 16, 16)]
        plsc.addupdate_scatter(slab, [kkvecs[kk], sidx], val)

    def compute(b):
      for j in range(CHUNK // 16):
        step(b, j)

    start(jnp.int32(0), 0)

    zeros16 = jnp.zeros((16,), jnp.float32)

    def zrow(i, carry):
      for kk in range(FS):
        slab[kk, pl.ds(i * 16, 16)] = zeros16
      return carry

    lax.fori_loop(0, NROW // 16, zrow, 0)

    def body(g, carry):
      k0 = g * 2
      k1 = k0 + 1

      @pl.when(k1 < m)
      def _():
        start(k1, 1)

      @pl.when(k0 < m)
      def _():
        wait(0)
        compute(0)

      @pl.when(k0 + 2 < m)
      def _():
        start(k0 + 2, 0)

      @pl.when(k1 < m)
      def _():
        wait(1)
        compute(1)

      return carry

    lax.fori_loop(0, npair, body, 0)

    if tail:
      base = n - CHUNK

      @pl.when(pg == tail_pg)
      def _():
        xb, yb, zb, fb, _ = bufs[0]
        pltpu.sync_copy(xs_h.at[pl.ds(base, CHUNK)], xb)
        pltpu.sync_copy(ys_h.at[pl.ds(base, CHUNK)], yb)
        pltpu.sync_copy(zs_h.at[pl.ds(base, CHUNK)], zb)
        for kk in range(FS):
          pltpu.sync_copy(ftf_h.at[pl.ds((col0 + kk) * n + base, CHUNK)],
                          fb.at[pl.ds(kk * CHUNK, CHUNK)])
        for j in range((CHUNK - tail) // 16, CHUNK // 16):
          step(0, j)

    pltpu.sync_copy(slab, out_h.at[pg, pl.ds(col0, FS), :])

  return k(xs, ys, zs, ftf)


def _combine_body(p_ref, o_ref):
  a = p_ref[0]
  for pg in range(1, NPG):
    a = a + p_ref[pg]
  o_ref[...] = a.T


def kernel(coords, features):
  b, n, _ = coords.shape
  f = features.shape[2]
  xs = coords[0, :, 0]
  ys = coords[0, :, 1]
  zs = coords[0, :, 2]
  ftf = features[0].T.reshape(f * n)
  partials = _sc_scatter(xs, ys, zs, ftf, n, f)
  summed = pl.pallas_call(
      _combine_body,
      out_shape=jax.ShapeDtypeStruct((NROW, f), jnp.float32),
  )(partials)
  return summed[:NCELL].reshape(b, BOX, BOX, BOX, f)
